# baseline (speedup 1.0000x reference)
import jax
import jax.numpy as jnp
from jax import lax
from jax.experimental import pallas as pl
from jax.experimental.pallas import tpu as pltpu
from jax.experimental.pallas import tpu_sc as plsc

F = 26
V1 = 100001
D = 32
B = 16384

NC = 2
NS = 16
L = 16
NW = NC * NS

CB = 4096
U = 8
FH = 13


def _body(xt_hbm, tabt_hbm, out_hbm, ib0, ib1, rowv, ob0, ob1,
          semr, semi0, semi1, semo0, semo1):
    s = lax.axis_index("s")
    d = s * NC + lax.axis_index("c")
    obs = (ob0, ob1)
    ibs = (ib0, ib1)
    semo = (semo0, semo1)
    semi = (semi0, semi1)

    def mega(base):
        def f_body(k, _):
            f = base + k
            cr = pltpu.async_copy(tabt_hbm.at[f, d], rowv, semr)
            cis = [None] * 4
            for h in (0, 1):
                cis[h] = pltpu.async_copy(
                    xt_hbm.at[f, pl.ds(h * CB, CB)], ibs[h], semi[h])
            cr.wait()

            for h in range(4):
                ob = obs[h % 2]
                ib = ibs[h % 2]
                cis[h].wait()

                @pl.when(f * 4 + h >= 2)
                def _(ob=ob, h=h):
                    pltpu.make_async_copy(
                        ob, out_hbm.at[f, d, pl.ds(h * CB, CB)],
                        semo[h % 2]).wait()

                def gstep(i, _, ib=ib, ob=ob):
                    base_i = i * (U * L)
                    idxs = [ib[pl.ds(base_i + u * L, L)] for u in range(U)]
                    vals = [plsc.load_gather(rowv, [ix]) for ix in idxs]
                    for u in range(U):
                        ob[pl.ds(base_i + u * L, L)] = vals[u]
                    return 0

                lax.fori_loop(0, CB // (U * L), gstep, 0)
                pltpu.async_copy(
                    ob, out_hbm.at[f, d, pl.ds(h * CB, CB)], semo[h % 2])
                if h + 2 < 4:
                    cis[h + 2] = pltpu.async_copy(
                        xt_hbm.at[f, pl.ds((h + 2) * CB, CB)],
                        ibs[h % 2], semi[h % 2])
            return 0

        lax.fori_loop(0, FH, f_body, 0)

    mega(0)
    mega(FH)
    for h in (2, 3):
        pltpu.make_async_copy(
            obs[h % 2], out_hbm.at[F - 1, d, pl.ds(h * CB, CB)],
            semo[h % 2]).wait()


@jax.jit
def kernel(x, tables):
    x_t = x.T
    tab_t = jnp.transpose(tables, (0, 2, 1))
    mesh = plsc.VectorSubcoreMesh(core_axis_name="c", subcore_axis_name="s",
                                  num_cores=NC, num_subcores=NS)
    out_t = pl.kernel(
        _body,
        out_type=jax.ShapeDtypeStruct((F, D, B), jnp.float32),
        mesh=mesh,
        scratch_types=[
            pltpu.VMEM((CB,), jnp.int32),
            pltpu.VMEM((CB,), jnp.int32),
            pltpu.VMEM((V1,), jnp.float32),
            pltpu.VMEM((CB,), jnp.float32),
            pltpu.VMEM((CB,), jnp.float32),
            pltpu.SemaphoreType.DMA,
            pltpu.SemaphoreType.DMA,
            pltpu.SemaphoreType.DMA,
            pltpu.SemaphoreType.DMA,
            pltpu.SemaphoreType.DMA,
        ],
        compiler_params=pltpu.CompilerParams(use_tc_tiling_on_sc=True,
                                             needs_layout_passes=False),
    )(x_t, tab_t)
    return jnp.transpose(out_t, (2, 0, 1))
